# Initial kernel scaffold; baseline (speedup 1.0000x reference)
import jax
import jax.numpy as jnp
from jax.experimental import pallas as pl


def kernel(x, table):
    raise NotImplementedError("write your pallas kernel here")


# speedup vs baseline: 1.5743x; 1.5743x over previous
import functools

import jax
import jax.numpy as jnp
from jax import lax
from jax.experimental import pallas as pl
from jax.experimental.pallas import tpu as pltpu
from jax.experimental.pallas import tpu_sc as plsc

_D = 64
_NC, _NS = 2, 16
_NW = _NC * _NS
_K = 128


@functools.cache
def _make_gather(B: int):
    b_per_w = B // _NW
    n_chunks = b_per_w // _K
    mesh = plsc.VectorSubcoreMesh(core_axis_name="c", subcore_axis_name="s")

    @functools.partial(
        pl.kernel,
        mesh=mesh,
        compiler_params=pltpu.CompilerParams(use_tc_tiling_on_sc=False),
        out_type=jax.ShapeDtypeStruct((B, _D), jnp.float32),
        scratch_types=[
            pltpu.VMEM((_K,), jnp.int32),
            pltpu.VMEM((_K, _D), jnp.float32),
            pltpu.SemaphoreType.DMA,
        ],
    )
    def gather_kernel(idx_hbm, table_hbm, out_hbm, idx_v, rows_v, sem):
        wid = lax.axis_index("s") * _NC + lax.axis_index("c")
        base = wid * b_per_w

        def body(g, _):
            off = base + g * _K
            pltpu.sync_copy(idx_hbm.at[pl.ds(off, _K)], idx_v)
            pltpu.async_copy(table_hbm.at[idx_v], rows_v, sem).wait()
            pltpu.sync_copy(rows_v, out_hbm.at[pl.ds(off, _K), :])
            return 0

        lax.fori_loop(0, n_chunks, body, 0)

    return gather_kernel


def kernel(x, table):
    b, h = x.shape
    idx = x.reshape(-1).astype(jnp.int32)
    out = _make_gather(b * h)(idx, table)
    return out.reshape(b, h, _D)


# speedup vs baseline: 1.8765x; 1.1919x over previous
import functools

import jax
import jax.numpy as jnp
from jax import lax
from jax.experimental import pallas as pl
from jax.experimental.pallas import tpu as pltpu
from jax.experimental.pallas import tpu_sc as plsc

_D = 64
_NC, _NS = 2, 16
_NW = _NC * _NS
_K = 128
_NBUF = 8
_AHEAD = 4


@functools.cache
def _make_gather(B: int):
    assert B % (_NW * _K) == 0
    b_per_w = B // _NW
    n_chunks = b_per_w // _K
    rounds = n_chunks // _NBUF
    assert n_chunks % _NBUF == 0 and rounds >= 3
    mesh = plsc.VectorSubcoreMesh(core_axis_name="c", subcore_axis_name="s")

    @functools.partial(
        pl.kernel,
        mesh=mesh,
        compiler_params=pltpu.CompilerParams(use_tc_tiling_on_sc=False),
        out_type=jax.ShapeDtypeStruct((B, _D), jnp.float32),
        scratch_types=(
            [pltpu.VMEM((n_chunks, _K), jnp.int32)]
            + [pltpu.VMEM((_K, _D), jnp.float32)] * _NBUF
            + [pltpu.SemaphoreType.DMA] * (2 * _NBUF)
        ),
    )
    def gather_kernel(idx_hbm, table_hbm, out_hbm, idx_v, *bufs):
        rows = bufs[:_NBUF]
        gsem = bufs[_NBUF:2 * _NBUF]
        wsem = bufs[2 * _NBUF:]
        wid = lax.axis_index("s") * _NC + lax.axis_index("c")
        chunk0 = wid * n_chunks
        base = wid * b_per_w

        pltpu.sync_copy(idx_hbm.at[pl.ds(chunk0, n_chunks)], idx_v)

        def start_gather(g, b):
            pltpu.async_copy(table_hbm.at[idx_v.at[g]], rows[b], gsem[b])

        def wait_gather(g, b):
            pltpu.make_async_copy(table_hbm.at[idx_v.at[g]], rows[b],
                                  gsem[b]).wait()

        def out_slice(g):
            return out_hbm.at[pl.ds(base + g * _K, _K)]

        def start_write(g, b):
            pltpu.async_copy(rows[b], out_slice(g), wsem[b])

        def wait_write(g, b):
            pltpu.make_async_copy(rows[b], out_slice(g), wsem[b]).wait()

        def step(g, b, do_wait_prev_write, do_next_gather):
            wait_gather(g, b)
            start_write(g, b)
            if do_next_gather:
                bn = (b + _AHEAD) % _NBUF
                if do_wait_prev_write:
                    wait_write(g + _AHEAD - _NBUF, bn)
                start_gather(g + _AHEAD, bn)

        for b in range(_AHEAD):
            start_gather(b, b)

        for b in range(_NBUF):
            step(b, b, do_wait_prev_write=(b + _AHEAD >= _NBUF),
                 do_next_gather=True)

        def mid_round(r, _):
            for b in range(_NBUF):
                step(r * _NBUF + b, b, True, True)
            return 0

        lax.fori_loop(1, rounds - 1, mid_round, 0)

        g_last = (rounds - 1) * _NBUF
        for b in range(_NBUF):
            step(g_last + b, b, do_wait_prev_write=(b < _AHEAD),
                 do_next_gather=(b < _AHEAD))

        for b in range(_NBUF):
            wait_write(n_chunks - _NBUF + b, b)

    return gather_kernel


def kernel(x, table):
    b, h = x.shape
    idx = x.reshape(-1, _K).astype(jnp.int32)
    out = _make_gather(b * h)(idx, table)
    return out.reshape(b, h, _D)
